# baseline (speedup 1.0000x reference)
import functools

import jax
import jax.numpy as jnp
from jax import lax
from jax.experimental import pallas as pl
from jax.experimental.pallas import tpu as pltpu
from jax.experimental.pallas import tpu_sc as plsc

N = 10000
D = 128
H = 32
E = 320000

NC = 2
NS = 16
NW = NC * NS
GROUP = 128
EPW = 10240
G = EPW // GROUP
EPAD = NW * EPW
NDEG = 10240
NACC = 10240
RPT = NACC // NS
DPT = NDEG // NS
NBUF = 4

_mesh = plsc.VectorSubcoreMesh(core_axis_name="c", subcore_axis_name="s")


@functools.partial(
    pl.kernel,
    out_type=jax.ShapeDtypeStruct((NC, NDEG), jnp.float32),
    mesh=_mesh,
    scratch_types=[
        pltpu.VMEM((G, GROUP), jnp.int32),
        pltpu.VMEM((G, GROUP), jnp.float32),
        pltpu.VMEM((DPT,), jnp.float32),
        pltpu.VMEM_SHARED((NDEG,), jnp.float32),
    ],
)
def _sc_deg(col_hbm, ew_hbm, out_hbm, colv, ewv, zv, deg_sh):
    cid = lax.axis_index("c")
    sid = lax.axis_index("s")
    wid = sid * NC + cid
    zero = jnp.zeros((16,), jnp.float32)

    def zbody(i, _):
        zv[pl.ds(i * 16, 16)] = zero
        return 0

    lax.fori_loop(0, DPT // 16, zbody, 0)
    pltpu.sync_copy(zv, deg_sh.at[pl.ds(sid * DPT, DPT)])
    plsc.subcore_barrier()

    pltpu.sync_copy(col_hbm.at[wid], colv)
    pltpu.sync_copy(ew_hbm.at[wid], ewv)

    def gbody(g, _):
        pltpu.sync_copy(ewv.at[g], deg_sh.at[colv.at[g]], add=True)
        return 0

    lax.fori_loop(0, G, gbody, 0)
    plsc.subcore_barrier()
    pltpu.sync_copy(deg_sh.at[pl.ds(sid * DPT, DPT)],
                    out_hbm.at[cid, pl.ds(sid * DPT, DPT)])


@functools.partial(
    pl.kernel,
    out_type=jax.ShapeDtypeStruct((NC, NACC, H), jnp.float32),
    mesh=_mesh,
    scratch_types=[
        pltpu.VMEM((G, GROUP), jnp.int32),
        pltpu.VMEM((G, GROUP), jnp.int32),
        pltpu.VMEM((G, GROUP), jnp.float32),
        pltpu.VMEM((NBUF, GROUP, H), jnp.float32),
        pltpu.VMEM((RPT, H), jnp.float32),
        pltpu.VMEM_SHARED((NACC, H), jnp.float32),
        pltpu.VMEM_SHARED((NACC, H), jnp.float32),
        [pltpu.SemaphoreType.DMA] * NBUF,
        [pltpu.SemaphoreType.DMA] * NBUF,
    ],
    compiler_params=pltpu.CompilerParams(use_tc_tiling_on_sc=False),
)
def _sc_agg(row_hbm, col_hbm, ew_hbm, u_hbm, out_hbm,
            rowv, colv, ewv, rows, zbuf, acc_sh, u_sh, gsem, ssem):
    cid = lax.axis_index("c")
    sid = lax.axis_index("s")
    wid = sid * NC + cid
    zero = jnp.zeros((16,), jnp.float32)

    def zbody(i, _):
        zbuf[i, pl.ds(0, 16)] = zero
        zbuf[i, pl.ds(16, 16)] = zero
        return 0

    lax.fori_loop(0, RPT, zbody, 0)
    pltpu.sync_copy(zbuf, acc_sh.at[pl.ds(sid * RPT, RPT)])
    pltpu.sync_copy(u_hbm.at[pl.ds(sid * RPT, RPT)],
                    u_sh.at[pl.ds(sid * RPT, RPT)])
    plsc.subcore_barrier()

    pltpu.sync_copy(row_hbm.at[wid], rowv)
    pltpu.sync_copy(col_hbm.at[wid], colv)
    pltpu.sync_copy(ew_hbm.at[wid], ewv)

    def start_gather(g, b):
        pltpu.async_copy(u_sh.at[rowv.at[g]], rows.at[b], gsem[b])

    def wait_gather(b):
        pltpu.make_async_copy(u_sh.at[rowv.at[0]], rows.at[b], gsem[b]).wait()

    def start_scatter(g, b):
        pltpu.async_copy(rows.at[b], acc_sh.at[colv.at[g]], ssem[b], add=True)

    def wait_scatter(b):
        pltpu.make_async_copy(rows.at[b], acc_sh.at[colv.at[0]], ssem[b]).wait()

    def scale(g, b):
        @plsc.parallel_loop(0, GROUP // 16, unroll=2)
        def sblock(i):
            wv = ewv[g, pl.ds(i * 16, 16)]
            base = i * 16
            for j in range(16):
                w = wv[j]
                rows[b, base + j, pl.ds(0, 16)] = (
                    rows[b, base + j, pl.ds(0, 16)] * w)
                rows[b, base + j, pl.ds(16, 16)] = (
                    rows[b, base + j, pl.ds(16, 16)] * w)

    start_gather(0, 0)
    start_gather(1, 1)
    for b in range(NBUF):
        g = b
        nb = (b + 2) % NBUF
        if g >= 2:
            wait_scatter(nb)
        start_gather(g + 2, nb)
        wait_gather(b)
        scale(g, b)
        start_scatter(g, b)

    def quad(q, _):
        for b in range(NBUF):
            g = q * NBUF + b
            nb = (b + 2) % NBUF
            wait_scatter(nb)
            start_gather(g + 2, nb)
            wait_gather(b)
            scale(g, b)
            start_scatter(g, b)
        return 0

    lax.fori_loop(1, G // NBUF - 1, quad, 0)

    for b in range(NBUF):
        g = (G // NBUF - 1) * NBUF + b
        nb = (b + 2) % NBUF
        wait_scatter(nb)
        if g + 2 < G:
            start_gather(g + 2, nb)
        wait_gather(b)
        scale(g, b)
        start_scatter(g, b)
    wait_scatter((G - 2) % NBUF)
    wait_scatter((G - 1) % NBUF)
    plsc.subcore_barrier()
    pltpu.sync_copy(acc_sh.at[pl.ds(sid * RPT, RPT)],
                    out_hbm.at[cid, pl.ds(sid * RPT, RPT)])


def _tc_z1_body(x_ref, w_ref, o_ref):
    o_ref[...] = jnp.dot(x_ref[...], w_ref[...],
                         preferred_element_type=jnp.float32)


def _tc_prescale_body(deg_ref, z_ref, dinv_ref, u_ref):
    deg = deg_ref[:, 0:1] + deg_ref[:, 1:2] + 1.0
    dinv = lax.rsqrt(deg)
    dinv_ref[...] = dinv
    u_ref[:N] = z_ref[...] * dinv
    u_ref[N:] = jnp.zeros((NACC - N, H), jnp.float32)


def _tc_mid_body(a_ref, u_ref, dinv_ref, b1_ref, w2_ref, u2_ref):
    acc = a_ref[0, :N] + a_ref[1, :N] + u_ref[:N]
    dinv = dinv_ref[...]
    h1 = jnp.maximum(dinv * acc + b1_ref[...], 0.0)
    z2 = jnp.dot(h1, w2_ref[...], preferred_element_type=jnp.float32)
    u2_ref[:N] = dinv * z2
    u2_ref[N:] = jnp.zeros((NACC - N, H), jnp.float32)


def _tc_final_body(a_ref, u_ref, dinv_ref, b2_ref, wc_ref, bc_ref, o_ref):
    acc = a_ref[0, :N] + a_ref[1, :N] + u_ref[:N]
    h2 = dinv_ref[...] * acc + b2_ref[...]
    o_ref[...] = jnp.dot(h2, wc_ref[...],
                         preferred_element_type=jnp.float32) + bc_ref[...]


def kernel(x, edge_index, edge_attr, W1, b1, W2, b2, Wc, bc):
    row = edge_index[0].astype(jnp.int32)
    col = edge_index[1].astype(jnp.int32)
    ew = edge_attr.astype(jnp.float32)
    pad = EPAD - E
    row3 = jnp.pad(row, (0, pad)).reshape(NW, G, GROUP)
    col3 = jnp.pad(col, (0, pad)).reshape(NW, G, GROUP)
    ew3 = jnp.pad(ew, (0, pad)).reshape(NW, G, GROUP)

    degp = _sc_deg(col3, ew3)
    z1 = pl.pallas_call(
        _tc_z1_body,
        out_shape=jax.ShapeDtypeStruct((N, H), jnp.float32),
    )(x, W1)

    degT = degp[:, :N].T
    dinv, u1 = pl.pallas_call(
        _tc_prescale_body,
        out_shape=(jax.ShapeDtypeStruct((N, 1), jnp.float32),
                   jax.ShapeDtypeStruct((NACC, H), jnp.float32)),
    )(degT, z1)

    accp1 = _sc_agg(row3, col3, ew3, u1)
    u2 = pl.pallas_call(
        _tc_mid_body,
        out_shape=jax.ShapeDtypeStruct((NACC, H), jnp.float32),
    )(accp1, u1, dinv, b1.reshape(1, H), W2)

    accp2 = _sc_agg(row3, col3, ew3, u2)
    out = pl.pallas_call(
        _tc_final_body,
        out_shape=jax.ShapeDtypeStruct((N, 1), jnp.float32),
    )(accp2, u2, dinv, b2.reshape(1, H), Wc, bc.reshape(1, 1))
    return out


# speedup vs baseline: 1.0567x; 1.0567x over previous
import functools

import jax
import jax.numpy as jnp
from jax import lax
from jax.experimental import pallas as pl
from jax.experimental.pallas import tpu as pltpu
from jax.experimental.pallas import tpu_sc as plsc

N = 10000
D = 128
H = 32
E = 320000

NC = 2
NS = 16
NW = NC * NS
GROUP = 128
G = 78
EPW = G * GROUP
GTOT = E // GROUP
TAILG = NW * G
NDEG = 10240
NACC = 10240
RPT = NACC // NS
DPT = NDEG // NS
NBUF = 4

_mesh = plsc.VectorSubcoreMesh(core_axis_name="c", subcore_axis_name="s")


@functools.partial(
    pl.kernel,
    out_type=jax.ShapeDtypeStruct((NC, NDEG), jnp.float32),
    mesh=_mesh,
    scratch_types=[
        pltpu.VMEM((G, GROUP), jnp.int32),
        pltpu.VMEM((EPW,), jnp.float32),
        pltpu.VMEM((GROUP,), jnp.int32),
        pltpu.VMEM((GROUP,), jnp.float32),
        pltpu.VMEM((DPT,), jnp.float32),
        pltpu.VMEM_SHARED((NDEG,), jnp.float32),
    ],
    compiler_params=pltpu.CompilerParams(use_tc_tiling_on_sc=False),
)
def _sc_deg(col_hbm, ew_hbm, out_hbm, colv, ewv, colx, ewx, zv, deg_sh):
    cid = lax.axis_index("c")
    sid = lax.axis_index("s")
    wid = sid * NC + cid
    zero = jnp.zeros((16,), jnp.float32)

    def zbody(i, _):
        zv[pl.ds(i * 16, 16)] = zero
        return 0

    lax.fori_loop(0, DPT // 16, zbody, 0)
    pltpu.sync_copy(zv, deg_sh.at[pl.ds(sid * DPT, DPT)])
    plsc.subcore_barrier()

    pltpu.sync_copy(col_hbm.at[pl.ds(wid * G, G)], colv)
    pltpu.sync_copy(ew_hbm.at[pl.ds(wid * EPW, EPW)], ewv)

    def gbody(g, _):
        pltpu.sync_copy(ewv.at[pl.ds(g * GROUP, GROUP)],
                        deg_sh.at[colv.at[g]], add=True)
        return 0

    lax.fori_loop(0, G, gbody, 0)

    @pl.when(wid < GTOT - TAILG)
    def _tail():
        pltpu.sync_copy(col_hbm.at[TAILG + wid], colx)
        pltpu.sync_copy(ew_hbm.at[pl.ds(TAILG * GROUP + wid * GROUP, GROUP)],
                        ewx)
        pltpu.sync_copy(ewx, deg_sh.at[colx], add=True)

    plsc.subcore_barrier()
    pltpu.sync_copy(deg_sh.at[pl.ds(sid * DPT, DPT)],
                    out_hbm.at[cid, pl.ds(sid * DPT, DPT)])


@functools.partial(
    pl.kernel,
    out_type=jax.ShapeDtypeStruct((NC, NACC, H), jnp.float32),
    mesh=_mesh,
    scratch_types=[
        pltpu.VMEM((EPW,), jnp.int32),
        pltpu.VMEM((G, GROUP), jnp.int32),
        pltpu.VMEM((EPW,), jnp.float32),
        pltpu.VMEM((NBUF, GROUP, H), jnp.float32),
        pltpu.VMEM((GROUP,), jnp.int32),
        pltpu.VMEM((GROUP,), jnp.int32),
        pltpu.VMEM((GROUP,), jnp.float32),
        pltpu.VMEM((GROUP, H), jnp.float32),
        pltpu.VMEM((RPT, H), jnp.float32),
        pltpu.VMEM_SHARED((NACC, H), jnp.float32),
        pltpu.VMEM_SHARED((NACC, H), jnp.float32),
        [pltpu.SemaphoreType.DMA] * NBUF,
        [pltpu.SemaphoreType.DMA] * NBUF,
        pltpu.SemaphoreType.DMA,
    ],
    compiler_params=pltpu.CompilerParams(use_tc_tiling_on_sc=False),
)
def _sc_agg(row_hbm, col_hbm, ew_hbm, u_hbm, out_hbm,
            rowv, colv, ewv, rows, rowx, colx, ewx, rowsx, zbuf,
            acc_sh, u_sh, gsem, ssem, tsem):
    cid = lax.axis_index("c")
    sid = lax.axis_index("s")
    wid = sid * NC + cid
    zero = jnp.zeros((16,), jnp.float32)

    def zbody(i, _):
        zbuf[i, pl.ds(0, 16)] = zero
        zbuf[i, pl.ds(16, 16)] = zero
        return 0

    lax.fori_loop(0, RPT, zbody, 0)
    pltpu.sync_copy(zbuf, acc_sh.at[pl.ds(sid * RPT, RPT)])
    pltpu.sync_copy(u_hbm.at[pl.ds(sid * RPT, RPT)],
                    u_sh.at[pl.ds(sid * RPT, RPT)])
    plsc.subcore_barrier()

    pltpu.sync_copy(row_hbm.at[pl.ds(wid * EPW, EPW)], rowv)
    pltpu.sync_copy(col_hbm.at[pl.ds(wid * G, G)], colv)
    pltpu.sync_copy(ew_hbm.at[pl.ds(wid * EPW, EPW)], ewv)

    def start_gather(g, b):
        pltpu.async_copy(u_sh.at[rowv.at[pl.ds(g * GROUP, GROUP)]],
                         rows.at[b], gsem[b])

    def wait_gather(b):
        pltpu.make_async_copy(u_sh.at[rowv.at[pl.ds(0, GROUP)]],
                              rows.at[b], gsem[b]).wait()

    def start_scatter(g, b):
        pltpu.async_copy(rows.at[b], acc_sh.at[colv.at[g]], ssem[b], add=True)

    def wait_scatter(b):
        pltpu.make_async_copy(rows.at[b], acc_sh.at[colv.at[0]],
                              ssem[b]).wait()

    def scale(g, b):
        def sblock(i, _):
            wv = ewv[pl.ds(g * GROUP + i * 16, 16)]
            base = i * 16
            for j in range(16):
                w = wv[j]
                rows[b, base + j, pl.ds(0, 16)] = (
                    rows[b, base + j, pl.ds(0, 16)] * w)
                rows[b, base + j, pl.ds(16, 16)] = (
                    rows[b, base + j, pl.ds(16, 16)] * w)
            return 0

        lax.fori_loop(0, GROUP // 16, sblock, 0)

    start_gather(0, 0)
    start_gather(1, 1)
    for b in range(NBUF):
        g = b
        nb = (b + 2) % NBUF
        if g >= 2:
            wait_scatter(nb)
        start_gather(g + 2, nb)
        wait_gather(b)
        scale(g, b)
        start_scatter(g, b)

    def quad(q, _):
        for b in range(NBUF):
            g = q * NBUF + b
            nb = (b + 2) % NBUF
            wait_scatter(nb)
            start_gather(g + 2, nb)
            wait_gather(b)
            scale(g, b)
            start_scatter(g, b)
        return 0

    lax.fori_loop(1, 18, quad, 0)

    for b in range(NBUF):
        g = 72 + b
        nb = (b + 2) % NBUF
        wait_scatter(nb)
        if g + 2 < G:
            start_gather(g + 2, nb)
        wait_gather(b)
        scale(g, b)
        start_scatter(g, b)

    for b in range(2):
        g = 76 + b
        nb = (b + 2) % NBUF
        wait_scatter(nb)
        wait_gather(b)
        scale(g, b)
        start_scatter(g, b)
    wait_scatter(0)
    wait_scatter(1)

    @pl.when(wid < GTOT - TAILG)
    def _tail():
        tbase = TAILG * GROUP + wid * GROUP
        pltpu.sync_copy(row_hbm.at[pl.ds(tbase, GROUP)], rowx)
        pltpu.sync_copy(col_hbm.at[TAILG + wid], colx)
        pltpu.sync_copy(ew_hbm.at[pl.ds(tbase, GROUP)], ewx)
        pltpu.async_copy(u_sh.at[rowx], rowsx, tsem).wait()

        def sblock(i, _):
            wv = ewx[pl.ds(i * 16, 16)]
            base = i * 16
            for j in range(16):
                w = wv[j]
                rowsx[base + j, pl.ds(0, 16)] = (
                    rowsx[base + j, pl.ds(0, 16)] * w)
                rowsx[base + j, pl.ds(16, 16)] = (
                    rowsx[base + j, pl.ds(16, 16)] * w)
            return 0

        lax.fori_loop(0, GROUP // 16, sblock, 0)
        pltpu.sync_copy(rowsx, acc_sh.at[colx], add=True)

    plsc.subcore_barrier()
    pltpu.sync_copy(acc_sh.at[pl.ds(sid * RPT, RPT)],
                    out_hbm.at[cid, pl.ds(sid * RPT, RPT)])


def _tc_z1_body(x_ref, w_ref, o_ref):
    o_ref[...] = jnp.dot(x_ref[...], w_ref[...],
                         preferred_element_type=jnp.float32)


def _tc_prescale_body(deg_ref, z_ref, dinv_ref, u_ref):
    deg = deg_ref[:, 0:1] + deg_ref[:, 1:2] + 1.0
    dinv = lax.rsqrt(deg)
    dinv_ref[...] = dinv
    u_ref[:N] = z_ref[...] * dinv
    u_ref[N:] = jnp.zeros((NACC - N, H), jnp.float32)


def _tc_mid_body(a_ref, u_ref, dinv_ref, b1_ref, w2_ref, u2_ref):
    acc = a_ref[0, :N] + a_ref[1, :N] + u_ref[:N]
    dinv = dinv_ref[...]
    h1 = jnp.maximum(dinv * acc + b1_ref[...], 0.0)
    z2 = jnp.dot(h1, w2_ref[...], preferred_element_type=jnp.float32)
    u2_ref[:N] = dinv * z2
    u2_ref[N:] = jnp.zeros((NACC - N, H), jnp.float32)


def _tc_final_body(a_ref, u_ref, dinv_ref, b2_ref, wc_ref, bc_ref, o_ref):
    acc = a_ref[0, :N] + a_ref[1, :N] + u_ref[:N]
    h2 = dinv_ref[...] * acc + b2_ref[...]
    o_ref[...] = jnp.dot(h2, wc_ref[...],
                         preferred_element_type=jnp.float32) + bc_ref[...]


def kernel(x, edge_index, edge_attr, W1, b1, W2, b2, Wc, bc):
    row = edge_index[0].astype(jnp.int32)
    col2 = edge_index[1].astype(jnp.int32).reshape(GTOT, GROUP)
    ew = edge_attr.astype(jnp.float32)

    degp = _sc_deg(col2, ew)
    z1 = pl.pallas_call(
        _tc_z1_body,
        out_shape=jax.ShapeDtypeStruct((N, H), jnp.float32),
    )(x, W1)

    degT = degp[:, :N].T
    dinv, u1 = pl.pallas_call(
        _tc_prescale_body,
        out_shape=(jax.ShapeDtypeStruct((N, 1), jnp.float32),
                   jax.ShapeDtypeStruct((NACC, H), jnp.float32)),
    )(degT, z1)

    accp1 = _sc_agg(row, col2, ew, u1)
    u2 = pl.pallas_call(
        _tc_mid_body,
        out_shape=jax.ShapeDtypeStruct((NACC, H), jnp.float32),
    )(accp1, u1, dinv, b1.reshape(1, H), W2)

    accp2 = _sc_agg(row, col2, ew, u2)
    out = pl.pallas_call(
        _tc_final_body,
        out_shape=jax.ShapeDtypeStruct((N, 1), jnp.float32),
    )(accp2, u2, dinv, b2.reshape(1, H), Wc, bc.reshape(1, 1))
    return out


# speedup vs baseline: 1.0568x; 1.0001x over previous
import functools

import jax
import jax.numpy as jnp
from jax import lax
from jax.experimental import pallas as pl
from jax.experimental.pallas import tpu as pltpu
from jax.experimental.pallas import tpu_sc as plsc

N = 10000
D = 128
H = 32
E = 320000

NC = 2
NS = 16
NW = NC * NS
GROUP = 128
G = 78
EPW = G * GROUP
GTOT = E // GROUP
TAILG = NW * G
NDEG = 10240
NACC = 10240
RPT = NACC // NS
DPT = NDEG // NS
NBUF = 4

_mesh = plsc.VectorSubcoreMesh(core_axis_name="c", subcore_axis_name="s")


@functools.partial(
    pl.kernel,
    out_type=jax.ShapeDtypeStruct((NC, NDEG), jnp.float32),
    mesh=_mesh,
    scratch_types=[
        pltpu.VMEM((EPW,), jnp.int32),
        pltpu.VMEM((EPW,), jnp.float32),
        pltpu.VMEM((GROUP,), jnp.int32),
        pltpu.VMEM((GROUP,), jnp.float32),
        pltpu.VMEM((DPT,), jnp.float32),
        pltpu.VMEM_SHARED((NDEG,), jnp.float32),
    ],
    compiler_params=pltpu.CompilerParams(use_tc_tiling_on_sc=False),
)
def _sc_deg(col_hbm, ew_hbm, out_hbm, colv, ewv, colx, ewx, zv, deg_sh):
    cid = lax.axis_index("c")
    sid = lax.axis_index("s")
    wid = sid * NC + cid
    zero = jnp.zeros((16,), jnp.float32)

    def zbody(i, _):
        zv[pl.ds(i * 16, 16)] = zero
        return 0

    lax.fori_loop(0, DPT // 16, zbody, 0)
    pltpu.sync_copy(zv, deg_sh.at[pl.ds(sid * DPT, DPT)])
    plsc.subcore_barrier()

    pltpu.sync_copy(col_hbm.at[pl.ds(wid * EPW, EPW)], colv)
    pltpu.sync_copy(ew_hbm.at[pl.ds(wid * EPW, EPW)], ewv)

    def gbody(g, _):
        pltpu.sync_copy(ewv.at[pl.ds(g * GROUP, GROUP)],
                        deg_sh.at[colv.at[pl.ds(g * GROUP, GROUP)]], add=True)
        return 0

    lax.fori_loop(0, G, gbody, 0)

    @pl.when(wid < GTOT - TAILG)
    def _tail():
        pltpu.sync_copy(col_hbm.at[pl.ds(TAILG * GROUP + wid * GROUP, GROUP)],
                        colx)
        pltpu.sync_copy(ew_hbm.at[pl.ds(TAILG * GROUP + wid * GROUP, GROUP)],
                        ewx)
        pltpu.sync_copy(ewx, deg_sh.at[colx], add=True)

    plsc.subcore_barrier()
    pltpu.sync_copy(deg_sh.at[pl.ds(sid * DPT, DPT)],
                    out_hbm.at[cid, pl.ds(sid * DPT, DPT)])


@functools.partial(
    pl.kernel,
    out_type=jax.ShapeDtypeStruct((NC, NACC, H), jnp.float32),
    mesh=_mesh,
    scratch_types=[
        pltpu.VMEM((EPW,), jnp.int32),
        pltpu.VMEM((EPW,), jnp.int32),
        pltpu.VMEM((EPW,), jnp.float32),
        pltpu.VMEM((NBUF, GROUP, H), jnp.float32),
        pltpu.VMEM((GROUP,), jnp.int32),
        pltpu.VMEM((GROUP,), jnp.int32),
        pltpu.VMEM((GROUP,), jnp.float32),
        pltpu.VMEM((GROUP, H), jnp.float32),
        pltpu.VMEM((RPT, H), jnp.float32),
        pltpu.VMEM_SHARED((NACC, H), jnp.float32),
        pltpu.VMEM_SHARED((NACC, H), jnp.float32),
        [pltpu.SemaphoreType.DMA] * NBUF,
        [pltpu.SemaphoreType.DMA] * NBUF,
        pltpu.SemaphoreType.DMA,
    ],
    compiler_params=pltpu.CompilerParams(use_tc_tiling_on_sc=False),
)
def _sc_agg(row_hbm, col_hbm, ew_hbm, u_hbm, out_hbm,
            rowv, colv, ewv, rows, rowx, colx, ewx, rowsx, zbuf,
            acc_sh, u_sh, gsem, ssem, tsem):
    cid = lax.axis_index("c")
    sid = lax.axis_index("s")
    wid = sid * NC + cid
    zero = jnp.zeros((16,), jnp.float32)

    def zbody(i, _):
        zbuf[i, pl.ds(0, 16)] = zero
        zbuf[i, pl.ds(16, 16)] = zero
        return 0

    lax.fori_loop(0, RPT, zbody, 0)
    pltpu.sync_copy(zbuf, acc_sh.at[pl.ds(sid * RPT, RPT)])
    pltpu.sync_copy(u_hbm.at[pl.ds(sid * RPT, RPT)],
                    u_sh.at[pl.ds(sid * RPT, RPT)])
    plsc.subcore_barrier()

    pltpu.sync_copy(row_hbm.at[pl.ds(wid * EPW, EPW)], rowv)
    pltpu.sync_copy(col_hbm.at[pl.ds(wid * EPW, EPW)], colv)
    pltpu.sync_copy(ew_hbm.at[pl.ds(wid * EPW, EPW)], ewv)

    def start_gather(g, b):
        pltpu.async_copy(u_sh.at[rowv.at[pl.ds(g * GROUP, GROUP)]],
                         rows.at[b], gsem[b])

    def wait_gather(b):
        pltpu.make_async_copy(u_sh.at[rowv.at[pl.ds(0, GROUP)]],
                              rows.at[b], gsem[b]).wait()

    def start_scatter(g, b):
        pltpu.async_copy(rows.at[b], acc_sh.at[colv.at[pl.ds(g * GROUP, GROUP)]],
                         ssem[b], add=True)

    def wait_scatter(b):
        pltpu.make_async_copy(rows.at[b], acc_sh.at[colv.at[pl.ds(0, GROUP)]],
                              ssem[b]).wait()

    def scale(g, b):
        def sblock(i, _):
            wv = ewv[pl.ds(g * GROUP + i * 16, 16)]
            base = i * 16
            for j in range(16):
                w = wv[j]
                rows[b, base + j, pl.ds(0, 16)] = (
                    rows[b, base + j, pl.ds(0, 16)] * w)
                rows[b, base + j, pl.ds(16, 16)] = (
                    rows[b, base + j, pl.ds(16, 16)] * w)
            return 0

        lax.fori_loop(0, GROUP // 16, sblock, 0)

    start_gather(0, 0)
    start_gather(1, 1)
    for b in range(NBUF):
        g = b
        nb = (b + 2) % NBUF
        if g >= 2:
            wait_scatter(nb)
        start_gather(g + 2, nb)
        wait_gather(b)
        scale(g, b)
        start_scatter(g, b)

    def quad(q, _):
        for b in range(NBUF):
            g = q * NBUF + b
            nb = (b + 2) % NBUF
            wait_scatter(nb)
            start_gather(g + 2, nb)
            wait_gather(b)
            scale(g, b)
            start_scatter(g, b)
        return 0

    lax.fori_loop(1, 18, quad, 0)

    for b in range(NBUF):
        g = 72 + b
        nb = (b + 2) % NBUF
        wait_scatter(nb)
        if g + 2 < G:
            start_gather(g + 2, nb)
        wait_gather(b)
        scale(g, b)
        start_scatter(g, b)

    for b in range(2):
        g = 76 + b
        nb = (b + 2) % NBUF
        wait_scatter(nb)
        wait_gather(b)
        scale(g, b)
        start_scatter(g, b)
    wait_scatter(0)
    wait_scatter(1)

    @pl.when(wid < GTOT - TAILG)
    def _tail():
        tbase = TAILG * GROUP + wid * GROUP
        pltpu.sync_copy(row_hbm.at[pl.ds(tbase, GROUP)], rowx)
        pltpu.sync_copy(col_hbm.at[pl.ds(tbase, GROUP)], colx)
        pltpu.sync_copy(ew_hbm.at[pl.ds(tbase, GROUP)], ewx)
        pltpu.async_copy(u_sh.at[rowx], rowsx, tsem).wait()

        def sblock(i, _):
            wv = ewx[pl.ds(i * 16, 16)]
            base = i * 16
            for j in range(16):
                w = wv[j]
                rowsx[base + j, pl.ds(0, 16)] = (
                    rowsx[base + j, pl.ds(0, 16)] * w)
                rowsx[base + j, pl.ds(16, 16)] = (
                    rowsx[base + j, pl.ds(16, 16)] * w)
            return 0

        lax.fori_loop(0, GROUP // 16, sblock, 0)
        pltpu.sync_copy(rowsx, acc_sh.at[colx], add=True)

    plsc.subcore_barrier()
    pltpu.sync_copy(acc_sh.at[pl.ds(sid * RPT, RPT)],
                    out_hbm.at[cid, pl.ds(sid * RPT, RPT)])


def _tc_z1_body(x_ref, w_ref, o_ref):
    o_ref[...] = jnp.dot(x_ref[...], w_ref[...],
                         preferred_element_type=jnp.float32)


def _tc_prescale_body(deg_ref, z_ref, dinv_ref, u_ref):
    deg = deg_ref[:, 0:1] + deg_ref[:, 1:2] + 1.0
    dinv = lax.rsqrt(deg)
    dinv_ref[...] = dinv
    u_ref[:N] = z_ref[...] * dinv
    u_ref[N:] = jnp.zeros((NACC - N, H), jnp.float32)


def _tc_mid_body(a_ref, u_ref, dinv_ref, b1_ref, w2_ref, u2_ref):
    acc = a_ref[0, :N] + a_ref[1, :N] + u_ref[:N]
    dinv = dinv_ref[...]
    h1 = jnp.maximum(dinv * acc + b1_ref[...], 0.0)
    z2 = jnp.dot(h1, w2_ref[...], preferred_element_type=jnp.float32)
    u2_ref[:N] = dinv * z2
    u2_ref[N:] = jnp.zeros((NACC - N, H), jnp.float32)


def _tc_final_body(a_ref, u_ref, dinv_ref, b2_ref, wc_ref, bc_ref, o_ref):
    acc = a_ref[0, :N] + a_ref[1, :N] + u_ref[:N]
    h2 = dinv_ref[...] * acc + b2_ref[...]
    o_ref[...] = jnp.dot(h2, wc_ref[...],
                         preferred_element_type=jnp.float32) + bc_ref[...]


def kernel(x, edge_index, edge_attr, W1, b1, W2, b2, Wc, bc):
    row = edge_index[0].astype(jnp.int32)
    col2 = edge_index[1].astype(jnp.int32)
    ew = edge_attr.astype(jnp.float32)

    degp = _sc_deg(col2, ew)
    z1 = pl.pallas_call(
        _tc_z1_body,
        out_shape=jax.ShapeDtypeStruct((N, H), jnp.float32),
    )(x, W1)

    degT = degp[:, :N].T
    dinv, u1 = pl.pallas_call(
        _tc_prescale_body,
        out_shape=(jax.ShapeDtypeStruct((N, 1), jnp.float32),
                   jax.ShapeDtypeStruct((NACC, H), jnp.float32)),
    )(degT, z1)

    accp1 = _sc_agg(row, col2, ew, u1)
    u2 = pl.pallas_call(
        _tc_mid_body,
        out_shape=jax.ShapeDtypeStruct((NACC, H), jnp.float32),
    )(accp1, u1, dinv, b1.reshape(1, H), W2)

    accp2 = _sc_agg(row, col2, ew, u2)
    out = pl.pallas_call(
        _tc_final_body,
        out_shape=jax.ShapeDtypeStruct((N, 1), jnp.float32),
    )(accp2, u2, dinv, b2.reshape(1, H), Wc, bc.reshape(1, 1))
    return out
